# baseline (speedup 1.0000x reference)
import functools

import jax
import jax.numpy as jnp
from jax import lax
from jax.experimental import pallas as pl
from jax.experimental.pallas import tpu as pltpu
from jax.experimental.pallas import tpu_sc as plsc

N_NODES = 10000
N_EDGES = 320000
IN_FEATS = 128
N_HIDDEN = 128
N_CLASSES = 41

D = 48
NP = 10240
NW = 32
GRP = 256
EPT = N_EDGES // NW
GB = 40
NTILE = 16
RPT = NP // NTILE
TPT = N_NODES // NTILE
NBUF = 4


def _proj_body(x_ref, w0_ref, b0_ref, w1_ref, c_ref, o_ref):
    h = jnp.dot(x_ref[...], w0_ref[...], preferred_element_type=jnp.float32)
    h = jnp.maximum(h + b0_ref[...], 0.0)
    o_ref[...] = (
        jnp.dot(h, w1_ref[...], preferred_element_type=jnp.float32) + c_ref[...]
    )


@functools.cache
def _sc_segment_sum():
    @functools.partial(
        pl.kernel,
        out_type=jax.ShapeDtypeStruct((2, NP, D), jnp.float32),
        mesh=plsc.VectorSubcoreMesh(core_axis_name="c", subcore_axis_name="s"),
        compiler_params=pltpu.CompilerParams(use_tc_tiling_on_sc=False),
        scratch_types=[
            pltpu.VMEM((GB * GRP,), jnp.int32),
            pltpu.VMEM((GB * GRP,), jnp.int32),
            pltpu.VMEM((NBUF, GRP, D), jnp.float32),
            pltpu.VMEM_SHARED((NP, D), jnp.float32),
            pltpu.VMEM_SHARED((N_NODES, D), jnp.float32),
            pltpu.SemaphoreType.DMA((NBUF,)),
        ],
    )
    def body_fn(gtab_hbm, eidx_hbm, zeros_hbm, out_hbm,
                src_v, dst_v, rows_v, acc_sh, gtab_sh, sem):
        c = lax.axis_index("c")
        s = lax.axis_index("s")
        wid = c * NTILE + s
        e0 = wid * EPT

        for k in range(EPT, GB * GRP, 16):
            src_v[pl.ds(k, 16)] = jnp.zeros((16,), jnp.int32)
            dst_v[pl.ds(k, 16)] = jnp.full((16,), N_NODES, jnp.int32)

        pltpu.sync_copy(zeros_hbm, acc_sh.at[pl.ds(s * RPT, RPT)])
        pltpu.sync_copy(gtab_hbm.at[pl.ds(s * TPT, TPT)],
                        gtab_sh.at[pl.ds(s * TPT, TPT)])
        pltpu.sync_copy(eidx_hbm.at[0, pl.ds(e0, EPT)], src_v.at[pl.ds(0, EPT)])
        pltpu.sync_copy(eidx_hbm.at[1, pl.ds(e0, EPT)], dst_v.at[pl.ds(0, EPT)])
        plsc.subcore_barrier()

        for k in range(NBUF):
            pltpu.async_copy(gtab_sh.at[src_v.at[pl.ds(k * GRP, GRP)]],
                             rows_v.at[k], sem.at[k])

        def body(j, carry):
            b = lax.rem(j, NBUF)
            pltpu.make_async_copy(
                gtab_sh.at[src_v.at[pl.ds(j * GRP, GRP)]], rows_v.at[b],
                sem.at[b]).wait()
            pltpu.sync_copy(rows_v.at[b],
                            acc_sh.at[dst_v.at[pl.ds(j * GRP, GRP)]],
                            add=True)

            @pl.when(j + NBUF < GB)
            def _():
                pltpu.async_copy(
                    gtab_sh.at[src_v.at[pl.ds((j + NBUF) * GRP, GRP)]],
                    rows_v.at[b], sem.at[b])

            return carry

        lax.fori_loop(0, GB, body, 0)
        plsc.subcore_barrier()
        pltpu.sync_copy(acc_sh.at[pl.ds(s * RPT, RPT)],
                        out_hbm.at[c, pl.ds(s * RPT, RPT)])

    return body_fn


def _finalize_body(acc_ref, b1_ref, o_ref):
    t = acc_ref[0] + acc_ref[1]
    cnt = jnp.maximum(t[:, N_CLASSES:N_CLASSES + 1], 1.0)
    o_ref[...] = t[:, :N_CLASSES] / cnt + b1_ref[...]


def kernel(x, edge_index, W0, b0, W1, b1):
    eidx = edge_index.astype(jnp.int32)

    w0t = W0.T
    b0r = b0.reshape(1, N_HIDDEN)
    w1t = jnp.zeros((N_HIDDEN, D), jnp.float32).at[:, :N_CLASSES].set(W1.T)
    crow = jnp.zeros((1, D), jnp.float32).at[0, N_CLASSES].set(1.0)

    BM = 1000
    gtab = pl.pallas_call(
        _proj_body,
        grid=(N_NODES // BM,),
        in_specs=[
            pl.BlockSpec((BM, IN_FEATS), lambda i: (i, 0)),
            pl.BlockSpec((IN_FEATS, N_HIDDEN), lambda i: (0, 0)),
            pl.BlockSpec((1, N_HIDDEN), lambda i: (0, 0)),
            pl.BlockSpec((N_HIDDEN, D), lambda i: (0, 0)),
            pl.BlockSpec((1, D), lambda i: (0, 0)),
        ],
        out_specs=pl.BlockSpec((BM, D), lambda i: (i, 0)),
        out_shape=jax.ShapeDtypeStruct((N_NODES, D), jnp.float32),
    )(x, w0t, b0r, w1t, crow)

    zeros_blk = jnp.zeros((RPT, D), jnp.float32)
    acc2 = _sc_segment_sum()(gtab, eidx, zeros_blk)

    BF = 1000
    out = pl.pallas_call(
        _finalize_body,
        grid=(N_NODES // BF,),
        in_specs=[
            pl.BlockSpec((2, BF, D), lambda i: (0, i, 0)),
            pl.BlockSpec((1, N_CLASSES), lambda i: (0, 0)),
        ],
        out_specs=pl.BlockSpec((BF, N_CLASSES), lambda i: (i, 0)),
        out_shape=jax.ShapeDtypeStruct((N_NODES, N_CLASSES), jnp.float32),
    )(acc2, b1.reshape(1, N_CLASSES))
    return out


# speedup vs baseline: 1.0247x; 1.0247x over previous
import functools

import jax
import jax.numpy as jnp
from jax import lax
from jax.experimental import pallas as pl
from jax.experimental.pallas import tpu as pltpu
from jax.experimental.pallas import tpu_sc as plsc

N_NODES = 10000
N_EDGES = 320000
IN_FEATS = 128
N_HIDDEN = 128
N_CLASSES = 41

D = 48
NP = 10240
NW = 32
GRP = 128
EPT = N_EDGES // NW
GB = 79
NTILE = 16
RPT = NP // NTILE
TPT = N_NODES // NTILE
NBUF = 4


def _proj_body(x_ref, w0_ref, b0_ref, w1_ref, c_ref, o_ref):
    h = jnp.dot(x_ref[...], w0_ref[...], preferred_element_type=jnp.float32)
    h = jnp.maximum(h + b0_ref[...], 0.0)
    o_ref[...] = (
        jnp.dot(h, w1_ref[...], preferred_element_type=jnp.float32) + c_ref[...]
    )


@functools.cache
def _sc_segment_sum():
    @functools.partial(
        pl.kernel,
        out_type=jax.ShapeDtypeStruct((2, NP, D), jnp.float32),
        mesh=plsc.VectorSubcoreMesh(core_axis_name="c", subcore_axis_name="s"),
        compiler_params=pltpu.CompilerParams(use_tc_tiling_on_sc=False),
        scratch_types=[
            pltpu.VMEM((GB * GRP,), jnp.int32),
            pltpu.VMEM((GB * GRP,), jnp.int32),
            pltpu.VMEM((NBUF, GRP, D), jnp.float32),
            pltpu.VMEM_SHARED((NP, D), jnp.float32),
            pltpu.VMEM_SHARED((N_NODES, D), jnp.float32),
            pltpu.SemaphoreType.DMA((NBUF,)),
        ],
    )
    def body_fn(gtab_hbm, eidx_hbm, zeros_hbm, out_hbm,
                src_v, dst_v, rows_v, acc_sh, gtab_sh, sem):
        c = lax.axis_index("c")
        s = lax.axis_index("s")
        wid = c * NTILE + s
        e0 = wid * EPT

        for k in range(EPT, GB * GRP, 16):
            src_v[pl.ds(k, 16)] = jnp.zeros((16,), jnp.int32)
            dst_v[pl.ds(k, 16)] = jnp.full((16,), N_NODES, jnp.int32)

        pltpu.sync_copy(zeros_hbm, acc_sh.at[pl.ds(s * RPT, RPT)])
        pltpu.sync_copy(gtab_hbm.at[pl.ds(s * TPT, TPT)],
                        gtab_sh.at[pl.ds(s * TPT, TPT)])
        pltpu.sync_copy(eidx_hbm.at[0, pl.ds(e0, EPT)], src_v.at[pl.ds(0, EPT)])
        pltpu.sync_copy(eidx_hbm.at[1, pl.ds(e0, EPT)], dst_v.at[pl.ds(0, EPT)])
        plsc.subcore_barrier()

        for k in range(NBUF):
            pltpu.async_copy(gtab_sh.at[src_v.at[pl.ds(k * GRP, GRP)]],
                             rows_v.at[k], sem.at[k])

        def body(j, carry):
            b = lax.rem(j, NBUF)
            pltpu.make_async_copy(
                gtab_sh.at[src_v.at[pl.ds(j * GRP, GRP)]], rows_v.at[b],
                sem.at[b]).wait()
            pltpu.sync_copy(rows_v.at[b],
                            acc_sh.at[dst_v.at[pl.ds(j * GRP, GRP)]],
                            add=True)

            @pl.when(j + NBUF < GB)
            def _():
                pltpu.async_copy(
                    gtab_sh.at[src_v.at[pl.ds((j + NBUF) * GRP, GRP)]],
                    rows_v.at[b], sem.at[b])

            return carry

        lax.fori_loop(0, GB, body, 0)
        plsc.subcore_barrier()
        pltpu.sync_copy(acc_sh.at[pl.ds(s * RPT, RPT)],
                        out_hbm.at[c, pl.ds(s * RPT, RPT)])

    return body_fn


def _finalize_body(acc_hbm, b1_ref, o_ref, acc_v, semf):
    pltpu.async_copy(acc_hbm, acc_v, semf).wait()
    t = acc_v[0, :N_NODES, :] + acc_v[1, :N_NODES, :]
    cnt = jnp.maximum(t[:, N_CLASSES:N_CLASSES + 1], 1.0)
    o_ref[...] = t[:, :N_CLASSES] / cnt + b1_ref[...]


def kernel(x, edge_index, W0, b0, W1, b1):
    eidx = edge_index.astype(jnp.int32)

    w0t = W0.T
    b0r = b0.reshape(1, N_HIDDEN)
    w1t = jnp.zeros((N_HIDDEN, D), jnp.float32).at[:, :N_CLASSES].set(W1.T)
    crow = jnp.zeros((1, D), jnp.float32).at[0, N_CLASSES].set(1.0)

    BM = 1000
    gtab = pl.pallas_call(
        _proj_body,
        grid=(N_NODES // BM,),
        in_specs=[
            pl.BlockSpec((BM, IN_FEATS), lambda i: (i, 0)),
            pl.BlockSpec((IN_FEATS, N_HIDDEN), lambda i: (0, 0)),
            pl.BlockSpec((1, N_HIDDEN), lambda i: (0, 0)),
            pl.BlockSpec((N_HIDDEN, D), lambda i: (0, 0)),
            pl.BlockSpec((1, D), lambda i: (0, 0)),
        ],
        out_specs=pl.BlockSpec((BM, D), lambda i: (i, 0)),
        out_shape=jax.ShapeDtypeStruct((N_NODES, D), jnp.float32),
    )(x, w0t, b0r, w1t, crow)

    zeros_blk = jnp.zeros((RPT, D), jnp.float32)
    acc2 = _sc_segment_sum()(gtab, eidx, zeros_blk)

    out = pl.pallas_call(
        _finalize_body,
        in_specs=[
            pl.BlockSpec(memory_space=pl.ANY),
            pl.BlockSpec((1, N_CLASSES), lambda: (0, 0)),
        ],
        out_specs=pl.BlockSpec((N_NODES, N_CLASSES), lambda: (0, 0)),
        out_shape=jax.ShapeDtypeStruct((N_NODES, N_CLASSES), jnp.float32),
        scratch_shapes=[
            pltpu.VMEM((2, NP, D), jnp.float32),
            pltpu.SemaphoreType.DMA,
        ],
    )(acc2, b1.reshape(1, N_CLASSES))
    return out
